# baseline (speedup 1.0000x reference)
import functools

import jax
import jax.numpy as jnp
from jax import lax
from jax.experimental import pallas as pl
from jax.experimental.pallas import tpu as pltpu
from jax.experimental.pallas import tpu_sc as plsc

B, F = 16384, 100
N = B * F
V = 1000000
NC, NS, L = 2, 16, 16
NW = NC * NS
PER_W = N // NW
CHUNK = 2048
NCHUNK = PER_W // CHUNK


def _sc_body(idx_hbm, table_hbm, y_hbm, part_hbm,
             idx_v0, idx_v1, rows_v0, rows_v1, acc_v, gsem, isem, wsem):
    wid = lax.axis_index("s") * NC + lax.axis_index("c")
    base = wid * PER_W
    idx_bufs = [idx_v0, idx_v1]
    row_bufs = [rows_v0, rows_v1]

    def idx_start(ci):
        o = base + ci * CHUNK
        j = o // B
        i0 = o - j * B
        return pltpu.async_copy(
            idx_hbm.at[j, pl.ds(i0, CHUNK)], idx_bufs[ci % 2], isem[ci % 2])

    def gather_start(ci):
        b = ci % 2
        return pltpu.async_copy(
            table_hbm.at[idx_bufs[b]], row_bufs[b], gsem[b])

    def wb_start(ci):
        return pltpu.async_copy(
            row_bufs[ci % 2],
            y_hbm.at[pl.ds(base + ci * CHUNK, CHUNK)], wsem[ci % 2])

    idx_start(0).wait()
    g0 = gather_start(0)
    i1 = idx_start(1)
    acc = jnp.zeros((L,), jnp.float32)
    wbs = [None, None]
    for ci in range(NCHUNK):
        g0.wait()
        if ci + 1 < NCHUNK:
            i1.wait()
            if wbs[(ci + 1) % 2] is not None:
                wbs[(ci + 1) % 2].wait()
            g0 = gather_start(ci + 1)
        if ci + 2 < NCHUNK:
            i1 = idx_start(ci + 2)

        def vec_body(i, a, _buf=row_bufs[ci % 2]):
            v = _buf[pl.ds(i * L, L)]
            return a + jnp.maximum(v, 0.0)

        acc = lax.fori_loop(0, CHUNK // L, vec_body, acc)
        wbs[ci % 2] = wb_start(ci)
    wbs[0].wait()
    wbs[1].wait()
    acc_v[...] = acc
    pltpu.sync_copy(acc_v, part_hbm.at[wid])


_sc_gather = pl.kernel(
    _sc_body,
    out_type=[
        jax.ShapeDtypeStruct((N,), jnp.float32),
        jax.ShapeDtypeStruct((NW, L), jnp.float32),
    ],
    mesh=plsc.VectorSubcoreMesh(core_axis_name="c", subcore_axis_name="s"),
    scratch_types=[
        pltpu.VMEM((CHUNK,), jnp.int32),
        pltpu.VMEM((CHUNK,), jnp.int32),
        pltpu.VMEM((CHUNK,), jnp.float32),
        pltpu.VMEM((CHUNK,), jnp.float32),
        pltpu.VMEM((L,), jnp.float32),
        [pltpu.SemaphoreType.DMA, pltpu.SemaphoreType.DMA],
        [pltpu.SemaphoreType.DMA, pltpu.SemaphoreType.DMA],
        [pltpu.SemaphoreType.DMA, pltpu.SemaphoreType.DMA],
    ],
)

M, K = N // 128, 128
BM = 3200


def _tc_body(part_ref, y_ref, o_ref):
    inv = 1.0 / jnp.sum(part_ref[...])
    o_ref[...] = jnp.maximum(y_ref[...], 0.0) * inv


_tc_scale = pl.pallas_call(
    _tc_body,
    grid=(M // BM,),
    in_specs=[
        pl.BlockSpec((NW, L), lambda i: (0, 0)),
        pl.BlockSpec((BM, K), lambda i: (i, 0)),
    ],
    out_specs=pl.BlockSpec((BM, K), lambda i: (i, 0)),
    out_shape=jax.ShapeDtypeStruct((M, K), jnp.float32),
)


def kernel(x, weight):
    idx = jnp.transpose(x)
    table = jnp.transpose(weight).reshape(V)
    y, parts = _sc_gather(idx, table)
    out = _tc_scale(parts, y.reshape(M, K))
    return jnp.transpose(out.reshape(F, B, 1), (1, 0, 2))


# speedup vs baseline: 1.0883x; 1.0883x over previous
import functools

import jax
import jax.numpy as jnp
from jax import lax
from jax.experimental import pallas as pl
from jax.experimental.pallas import tpu as pltpu
from jax.experimental.pallas import tpu_sc as plsc

B, F = 16384, 100
N = B * F
V = 1000000
NC, NS, L = 2, 16, 16
NW = NC * NS
PER_W = N // NW
CHUNK = 2048
NCHUNK = PER_W // CHUNK


NBUF = 4
AHEAD_G = 3


def _sc_body(idx_hbm, table_hbm, y_hbm, part_hbm,
             idx_bufs, row_bufs, acc_v, gsem, isem, wsem):
    wid = lax.axis_index("s") * NC + lax.axis_index("c")
    base = wid * PER_W

    def idx_start(ci):
        o = base + ci * CHUNK
        j = o // B
        i0 = o - j * B
        return pltpu.async_copy(
            idx_hbm.at[j, pl.ds(i0, CHUNK)], idx_bufs[ci % NBUF],
            isem[ci % NBUF])

    def gather_start(ci):
        b = ci % NBUF
        return pltpu.async_copy(
            table_hbm.at[idx_bufs[b]], row_bufs[b], gsem[b])

    def wb_start(ci):
        return pltpu.async_copy(
            row_bufs[ci % NBUF],
            y_hbm.at[pl.ds(base + ci * CHUNK, CHUNK)], wsem[ci % NBUF])

    iops, gops, wops = {}, {}, {}

    def start_gather(k):
        iops[k].wait()
        if k - NBUF >= 0:
            wops[k - NBUF].wait()
        gops[k] = gather_start(k)

    for k in range(min(NBUF, NCHUNK)):
        iops[k] = idx_start(k)
    for k in range(min(AHEAD_G, NCHUNK)):
        start_gather(k)

    acc = jnp.zeros((L,), jnp.float32)
    for ci in range(NCHUNK):
        gops[ci].wait()
        if ci + AHEAD_G < NCHUNK:
            start_gather(ci + AHEAD_G)
        if ci + NBUF < NCHUNK:
            iops[ci + NBUF] = idx_start(ci + NBUF)

        def vec_body(i, a, _buf=row_bufs[ci % NBUF]):
            v = _buf[pl.ds(i * L, L)]
            return a + jnp.maximum(v, 0.0)

        acc = lax.fori_loop(0, CHUNK // L, vec_body, acc)
        wops[ci] = wb_start(ci)
    for k in range(max(0, NCHUNK - NBUF), NCHUNK):
        wops[k].wait()
    acc_v[...] = acc
    pltpu.sync_copy(acc_v, part_hbm.at[wid])


_sc_gather = pl.kernel(
    _sc_body,
    out_type=[
        jax.ShapeDtypeStruct((N,), jnp.float32),
        jax.ShapeDtypeStruct((NW, L), jnp.float32),
    ],
    mesh=plsc.VectorSubcoreMesh(core_axis_name="c", subcore_axis_name="s"),
    scratch_types=[
        [pltpu.VMEM((CHUNK,), jnp.int32) for _ in range(NBUF)],
        [pltpu.VMEM((CHUNK,), jnp.float32) for _ in range(NBUF)],
        pltpu.VMEM((L,), jnp.float32),
        [pltpu.SemaphoreType.DMA for _ in range(NBUF)],
        [pltpu.SemaphoreType.DMA for _ in range(NBUF)],
        [pltpu.SemaphoreType.DMA for _ in range(NBUF)],
    ],
)

M, K = N // 128, 128
BM = 3200


def _tc_body(part_ref, y_ref, o_ref):
    inv = 1.0 / jnp.sum(part_ref[...])
    o_ref[...] = jnp.maximum(y_ref[...], 0.0) * inv


_tc_scale = pl.pallas_call(
    _tc_body,
    grid=(M // BM,),
    in_specs=[
        pl.BlockSpec((NW, L), lambda i: (0, 0)),
        pl.BlockSpec((BM, K), lambda i: (i, 0)),
    ],
    out_specs=pl.BlockSpec((BM, K), lambda i: (i, 0)),
    out_shape=jax.ShapeDtypeStruct((M, K), jnp.float32),
)


def kernel(x, weight):
    idx = jnp.transpose(x)
    table = jnp.transpose(weight).reshape(V)
    y, parts = _sc_gather(idx, table)
    out = _tc_scale(parts, y.reshape(M, K))
    return jnp.transpose(out.reshape(F, B, 1), (1, 0, 2))


# speedup vs baseline: 1.0927x; 1.0040x over previous
import functools

import jax
import jax.numpy as jnp
from jax import lax
from jax.experimental import pallas as pl
from jax.experimental.pallas import tpu as pltpu
from jax.experimental.pallas import tpu_sc as plsc

B, F = 16384, 100
N = B * F
V = 1000000
NC, NS, L = 2, 16, 16
NW = NC * NS
PER_W = N // NW
CHUNK = 2048
NCHUNK = PER_W // CHUNK


NBUF = 6
AHEAD_G = 5


def _sc_body(idx_hbm, table_hbm, y_hbm, part_hbm,
             idx_bufs, row_bufs, acc_v, gsem, isem, wsem):
    wid = lax.axis_index("s") * NC + lax.axis_index("c")
    base = wid * PER_W

    def idx_start(ci):
        o = base + ci * CHUNK
        j = o // B
        i0 = o - j * B
        return pltpu.async_copy(
            idx_hbm.at[j, pl.ds(i0, CHUNK)], idx_bufs[ci % NBUF],
            isem[ci % NBUF])

    def gather_start(ci):
        b = ci % NBUF
        return pltpu.async_copy(
            table_hbm.at[idx_bufs[b]], row_bufs[b], gsem[b])

    def wb_start(ci):
        return pltpu.async_copy(
            row_bufs[ci % NBUF],
            y_hbm.at[pl.ds(base + ci * CHUNK, CHUNK)], wsem[ci % NBUF])

    iops, gops, wops = {}, {}, {}

    def start_gather(k):
        iops[k].wait()
        if k - NBUF >= 0:
            wops[k - NBUF].wait()
        gops[k] = gather_start(k)

    for k in range(min(NBUF, NCHUNK)):
        iops[k] = idx_start(k)
    for k in range(min(AHEAD_G, NCHUNK)):
        start_gather(k)

    acc = jnp.zeros((L,), jnp.float32)
    for ci in range(NCHUNK):
        gops[ci].wait()
        if ci + AHEAD_G < NCHUNK:
            start_gather(ci + AHEAD_G)
        if ci + NBUF < NCHUNK:
            iops[ci + NBUF] = idx_start(ci + NBUF)

        def vec_body(i, a, _buf=row_bufs[ci % NBUF]):
            v = _buf[pl.ds(i * L, L)]
            return a + jnp.maximum(v, 0.0)

        acc = lax.fori_loop(0, CHUNK // L, vec_body, acc)
        wops[ci] = wb_start(ci)
    for k in range(max(0, NCHUNK - NBUF), NCHUNK):
        wops[k].wait()
    acc_v[...] = acc
    pltpu.sync_copy(acc_v, part_hbm.at[wid])


_sc_gather = pl.kernel(
    _sc_body,
    out_type=[
        jax.ShapeDtypeStruct((N,), jnp.float32),
        jax.ShapeDtypeStruct((NW, L), jnp.float32),
    ],
    mesh=plsc.VectorSubcoreMesh(core_axis_name="c", subcore_axis_name="s"),
    scratch_types=[
        [pltpu.VMEM((CHUNK,), jnp.int32) for _ in range(NBUF)],
        [pltpu.VMEM((CHUNK,), jnp.float32) for _ in range(NBUF)],
        pltpu.VMEM((L,), jnp.float32),
        [pltpu.SemaphoreType.DMA for _ in range(NBUF)],
        [pltpu.SemaphoreType.DMA for _ in range(NBUF)],
        [pltpu.SemaphoreType.DMA for _ in range(NBUF)],
    ],
)

M, K = N // 128, 128
BM = 3200


def _tc_body(part_ref, y_ref, o_ref):
    inv = 1.0 / jnp.sum(part_ref[...])
    o_ref[...] = jnp.maximum(y_ref[...], 0.0) * inv


_tc_scale = pl.pallas_call(
    _tc_body,
    grid=(M // BM,),
    in_specs=[
        pl.BlockSpec((NW, L), lambda i: (0, 0)),
        pl.BlockSpec((BM, K), lambda i: (i, 0)),
    ],
    out_specs=pl.BlockSpec((BM, K), lambda i: (i, 0)),
    out_shape=jax.ShapeDtypeStruct((M, K), jnp.float32),
)


def kernel(x, weight):
    idx = jnp.transpose(x)
    table = jnp.transpose(weight).reshape(V)
    y, parts = _sc_gather(idx, table)
    out = _tc_scale(parts, y.reshape(M, K))
    return jnp.transpose(out.reshape(F, B, 1), (1, 0, 2))


# speedup vs baseline: 1.4850x; 1.3591x over previous
import functools

import jax
import jax.numpy as jnp
from jax import lax
from jax.experimental import pallas as pl
from jax.experimental.pallas import tpu as pltpu
from jax.experimental.pallas import tpu_sc as plsc

B, F = 16384, 100
N = B * F
V = 1000000
NC, NS, L = 2, 16, 16
NW = NC * NS
PER_W = N // NW
CHUNK = 2048
NCHUNK = PER_W // CHUNK


NBUF = 6
AHEAD_G = 5


def _sc_body(idx_hbm, table_hbm, y_hbm, part_hbm,
             idx_bufs, row_bufs, acc_v, gsem, isem, wsem):
    wid = lax.axis_index("s") * NC + lax.axis_index("c")
    base = wid * PER_W

    def idx_start(ci):
        o = base + ci * CHUNK
        j = o // B
        i0 = o - j * B
        return pltpu.async_copy(
            idx_hbm.at[j, pl.ds(i0, CHUNK)], idx_bufs[ci % NBUF],
            isem[ci % NBUF])

    tb = table_hbm.at[0]

    def gather_start(ci):
        b = ci % NBUF
        return pltpu.async_copy(
            tb.at[idx_bufs[b]], row_bufs[b], gsem[b])

    def wb_start(ci):
        return pltpu.async_copy(
            row_bufs[ci % NBUF],
            y_hbm.at[pl.ds(base + ci * CHUNK, CHUNK)], wsem[ci % NBUF])

    iops, gops, wops = {}, {}, {}

    def start_gather(k):
        iops[k].wait()
        if k - NBUF >= 0:
            wops[k - NBUF].wait()
        gops[k] = gather_start(k)

    for k in range(min(NBUF, NCHUNK)):
        iops[k] = idx_start(k)
    for k in range(min(AHEAD_G, NCHUNK)):
        start_gather(k)

    acc = jnp.zeros((L,), jnp.float32)
    for ci in range(NCHUNK):
        gops[ci].wait()
        if ci + AHEAD_G < NCHUNK:
            start_gather(ci + AHEAD_G)
        if ci + NBUF < NCHUNK:
            iops[ci + NBUF] = idx_start(ci + NBUF)

        def vec_body(i, a, _buf=row_bufs[ci % NBUF]):
            v = _buf[pl.ds(i * L, L)]
            return a + jnp.maximum(v, 0.0)

        acc = lax.fori_loop(0, CHUNK // L, vec_body, acc)
        wops[ci] = wb_start(ci)
    for k in range(max(0, NCHUNK - NBUF), NCHUNK):
        wops[k].wait()
    acc_v[...] = acc
    pltpu.sync_copy(acc_v, part_hbm.at[wid])


_sc_gather = pl.kernel(
    _sc_body,
    out_type=[
        jax.ShapeDtypeStruct((N,), jnp.float32),
        jax.ShapeDtypeStruct((NW, L), jnp.float32),
    ],
    mesh=plsc.VectorSubcoreMesh(core_axis_name="c", subcore_axis_name="s"),
    scratch_types=[
        [pltpu.VMEM((CHUNK,), jnp.int32) for _ in range(NBUF)],
        [pltpu.VMEM((CHUNK,), jnp.float32) for _ in range(NBUF)],
        pltpu.VMEM((L,), jnp.float32),
        [pltpu.SemaphoreType.DMA for _ in range(NBUF)],
        [pltpu.SemaphoreType.DMA for _ in range(NBUF)],
        [pltpu.SemaphoreType.DMA for _ in range(NBUF)],
    ],
)

M, K = N // 128, 128
BM = 3200


def _tc_body(part_ref, y_ref, o_ref):
    inv = 1.0 / jnp.sum(part_ref[...])
    o_ref[...] = jnp.maximum(y_ref[...], 0.0) * inv


_tc_scale = pl.pallas_call(
    _tc_body,
    grid=(M // BM,),
    in_specs=[
        pl.BlockSpec((NW, L), lambda i: (0, 0)),
        pl.BlockSpec((BM, K), lambda i: (i, 0)),
    ],
    out_specs=pl.BlockSpec((BM, K), lambda i: (i, 0)),
    out_shape=jax.ShapeDtypeStruct((M, K), jnp.float32),
)


def kernel(x, weight):
    idx = jnp.transpose(x)
    table = jnp.transpose(weight)
    y, parts = _sc_gather(idx, table)
    out = _tc_scale(parts, y.reshape(M, K))
    return jnp.transpose(out.reshape(F, B, 1), (1, 0, 2))
